# baseline (device time: 138953 ns/iter reference)
import jax
import jax.numpy as jnp
from jax import lax
from jax.experimental import pallas as pl
from jax.experimental.pallas import tpu as pltpu

N_DEV = 4
HQ = 8
DH = 128
SQ = 2048
D_MODEL = 1024
WINDOW = 128
SCALE = 0.08838834764831843
QBLK = 256
KWIN = QBLK + 2 * WINDOW
CHUNK = SQ // N_DEV
HALF = D_MODEL // 2
N_HOPS = 2 * (N_DEV - 1)


def kernel(x, Wq, K_ext, V_ext, Wo):
    r = lax.axis_index("i")
    K = lax.dynamic_slice_in_dim(K_ext[0], r * HQ, HQ, axis=1)
    V = lax.dynamic_slice_in_dim(V_ext[0], r * HQ, HQ, axis=1)
    xb = x[0].astype(jnp.bfloat16)
    wqb = Wq.astype(jnp.bfloat16)
    wob = Wo.astype(jnp.bfloat16)
    kb = K.astype(jnp.bfloat16)
    vb = V.astype(jnp.bfloat16)

    def body(x_ref, wq_ref, k_ref, v_ref, wo_ref, out_ref,
             ctx_ref, comm_ref, sbuf_ref, send_sems, recv_sems):
        my = lax.axis_index("i")
        left = (my + N_DEV - 1) % N_DEV
        right = (my + 1) % N_DEV

        barrier = pltpu.get_barrier_semaphore()
        for nbr in (left, right):
            pl.semaphore_signal(barrier, inc=1, device_id=(nbr,),
                                device_id_type=pl.DeviceIdType.MESH)
        pl.semaphore_wait(barrier, 2)

        def compute_chunk(c):
            q0 = c * CHUNK
            qc = jnp.dot(x_ref[pl.ds(q0, CHUNK), :], wq_ref[...],
                         preferred_element_type=jnp.float32)
            qc = (qc * SCALE).astype(jnp.bfloat16)
            for h in range(HQ):
                for sub in range(CHUNK // QBLK):
                    r0 = q0 + sub * QBLK
                    kw = jnp.minimum(jnp.maximum(r0 - WINDOW, 0), SQ - KWIN)
                    ks = k_ref[pl.ds(kw, KWIN), h, :]
                    vs = v_ref[pl.ds(kw, KWIN), h, :]
                    qs = qc[sub * QBLK:(sub + 1) * QBLK,
                            h * DH:(h + 1) * DH]
                    s = lax.dot_general(
                        qs, ks, (((1,), (1,)), ((), ())),
                        preferred_element_type=jnp.float32)
                    ii = r0 + lax.broadcasted_iota(jnp.int32, s.shape, 0)
                    jj = kw + lax.broadcasted_iota(jnp.int32, s.shape, 1)
                    s = jnp.where(jnp.abs(ii - jj) <= WINDOW, s, -1e9)
                    mx = jnp.max(s, axis=1, keepdims=True)
                    w = jnp.exp(s - mx)
                    p = (w / jnp.sum(w, axis=1, keepdims=True)).astype(
                        jnp.bfloat16)
                    ctx = jnp.dot(p, vs, preferred_element_type=jnp.float32)
                    ctx_ref[pl.ds(r0, QBLK), h * DH:(h + 1) * DH] = (
                        ctx.astype(jnp.bfloat16))
            out_ref[0, pl.ds(q0, CHUNK), :] = jnp.dot(
                ctx_ref[pl.ds(q0, CHUNK), :], wo_ref[...],
                preferred_element_type=jnp.float32)

        def ring_rdma(ring, hop, src):
            dev = right if ring == 0 else left
            return pltpu.make_async_remote_copy(
                src_ref=src,
                dst_ref=comm_ref.at[ring, hop],
                send_sem=send_sems.at[ring, hop],
                recv_sem=recv_sems.at[ring, hop],
                device_id=(dev,),
                device_id_type=pl.DeviceIdType.MESH)

        def stage_and_start(s_):
            cs0 = (my - s_ + N_DEV) % N_DEV
            cs1 = (my + s_) % N_DEV
            sbuf_ref[0] = out_ref[0, pl.ds(cs0 * CHUNK, CHUNK),
                                  0:HALF].astype(jnp.bfloat16)
            sbuf_ref[1] = out_ref[0, pl.ds(cs1 * CHUNK, CHUNK),
                                  HALF:D_MODEL].astype(jnp.bfloat16)
            r0 = ring_rdma(0, s_, sbuf_ref.at[0])
            r1 = ring_rdma(1, s_, sbuf_ref.at[1])
            r0.start()
            r1.start()
            return r0, r1

        def wait_and_add(s_, r0, r1):
            r0.wait()
            r1.wait()
            cr0 = (my - s_ - 1 + N_DEV) % N_DEV
            cr1 = (my + s_ + 1) % N_DEV
            out_ref[0, pl.ds(cr0 * CHUNK, CHUNK), 0:HALF] = (
                out_ref[0, pl.ds(cr0 * CHUNK, CHUNK), 0:HALF]
                + comm_ref[0, s_].astype(jnp.float32))
            out_ref[0, pl.ds(cr1 * CHUNK, CHUNK), HALF:D_MODEL] = (
                out_ref[0, pl.ds(cr1 * CHUNK, CHUNK), HALF:D_MODEL]
                + comm_ref[1, s_].astype(jnp.float32))

        compute_chunk(my)
        h0 = stage_and_start(0)
        compute_chunk((my + 1) % N_DEV)
        compute_chunk((my + N_DEV - 1) % N_DEV)
        wait_and_add(0, *h0)
        h1 = stage_and_start(1)
        compute_chunk((my + 2) % N_DEV)
        wait_and_add(1, *h1)
        h2 = stage_and_start(2)
        wait_and_add(2, *h2)

        own0 = (my + 1) % N_DEV
        own1 = (my + N_DEV - 1) % N_DEV
        for g in range(N_DEV - 1):
            hop = (N_DEV - 1) + g
            if g == 0:
                sbuf_ref[0] = out_ref[0, pl.ds(own0 * CHUNK, CHUNK),
                                      0:HALF].astype(jnp.bfloat16)
                sbuf_ref[1] = out_ref[0, pl.ds(own1 * CHUNK, CHUNK),
                                      HALF:D_MODEL].astype(jnp.bfloat16)
                src0, src1 = sbuf_ref.at[0], sbuf_ref.at[1]
            else:
                src0 = comm_ref.at[0, hop - 1]
                src1 = comm_ref.at[1, hop - 1]
            r0 = ring_rdma(0, hop, src0)
            r1 = ring_rdma(1, hop, src1)
            r0.start()
            r1.start()
            r0.wait()
            r1.wait()
            cr0 = (my - g + N_DEV) % N_DEV
            cr1 = (my + g) % N_DEV
            out_ref[0, pl.ds(cr0 * CHUNK, CHUNK), 0:HALF] = (
                comm_ref[0, hop].astype(jnp.float32))
            out_ref[0, pl.ds(cr1 * CHUNK, CHUNK), HALF:D_MODEL] = (
                comm_ref[1, hop].astype(jnp.float32))

    out_shape = jax.ShapeDtypeStruct((1, SQ, D_MODEL), jnp.float32)
    return pl.pallas_call(
        body,
        out_shape=out_shape,
        in_specs=[pl.BlockSpec(memory_space=pltpu.VMEM)] * 5,
        out_specs=pl.BlockSpec(memory_space=pltpu.VMEM),
        scratch_shapes=[
            pltpu.VMEM((SQ, HQ * DH), jnp.bfloat16),
            pltpu.VMEM((2, N_HOPS, CHUNK, HALF), jnp.bfloat16),
            pltpu.VMEM((2, CHUNK, HALF), jnp.bfloat16),
            pltpu.SemaphoreType.DMA((2, N_HOPS)),
            pltpu.SemaphoreType.DMA((2, N_HOPS)),
        ],
        compiler_params=pltpu.CompilerParams(
            collective_id=0, vmem_limit_bytes=56 * 1024 * 1024),
    )(xb, wqb, kb, vb, wob)


# device time: 125619 ns/iter; 1.1061x vs baseline; 1.1061x over previous
import jax
import jax.numpy as jnp
from jax import lax
from jax.experimental import pallas as pl
from jax.experimental.pallas import tpu as pltpu

N_DEV = 4
HQ = 8
DH = 128
SQ = 2048
D_MODEL = 1024
WINDOW = 128
SCALE = 0.08838834764831843
QBLK = 512
KWIN = QBLK + 2 * WINDOW
CHUNK = SQ // N_DEV
HALF = D_MODEL // 2
N_HOPS = 2 * (N_DEV - 1)


def kernel(x, Wq, K_ext, V_ext, Wo):
    r = lax.axis_index("i")
    K = lax.dynamic_slice_in_dim(K_ext[0], r * HQ, HQ, axis=1)
    V = lax.dynamic_slice_in_dim(V_ext[0], r * HQ, HQ, axis=1)
    xb = x[0].astype(jnp.bfloat16)
    wqb = Wq.astype(jnp.bfloat16)
    wob = Wo.astype(jnp.bfloat16)
    kb = K.astype(jnp.bfloat16)
    vb = V.astype(jnp.bfloat16)

    def body(x_ref, wq_ref, k_ref, v_ref, wo_ref, out_ref,
             ctx_ref, comm_ref, sbuf_ref, send_sems, recv_sems):
        my = lax.axis_index("i")
        left = (my + N_DEV - 1) % N_DEV
        right = (my + 1) % N_DEV

        barrier = pltpu.get_barrier_semaphore()
        for nbr in (left, right):
            pl.semaphore_signal(barrier, inc=1, device_id=(nbr,),
                                device_id_type=pl.DeviceIdType.MESH)
        pl.semaphore_wait(barrier, 2)

        def compute_chunk(c):
            q0 = c * CHUNK
            qc = jnp.dot(x_ref[pl.ds(q0, CHUNK), :], wq_ref[...],
                         preferred_element_type=jnp.float32)
            qc = (qc * SCALE).astype(jnp.bfloat16)
            for h in range(HQ):
                for sub in range(CHUNK // QBLK):
                    r0 = q0 + sub * QBLK
                    kw = jnp.minimum(jnp.maximum(r0 - WINDOW, 0), SQ - KWIN)
                    ks = k_ref[pl.ds(kw, KWIN), h, :]
                    vs = v_ref[pl.ds(kw, KWIN), h, :]
                    qs = qc[sub * QBLK:(sub + 1) * QBLK,
                            h * DH:(h + 1) * DH]
                    s = lax.dot_general(
                        qs, ks, (((1,), (1,)), ((), ())),
                        preferred_element_type=jnp.float32)
                    ii = r0 + lax.broadcasted_iota(jnp.int32, s.shape, 0)
                    jj = kw + lax.broadcasted_iota(jnp.int32, s.shape, 1)
                    s = jnp.where(jnp.abs(ii - jj) <= WINDOW, s, -1e9)
                    mx = jnp.max(s, axis=1, keepdims=True)
                    w = jnp.exp(s - mx)
                    p = (w / jnp.sum(w, axis=1, keepdims=True)).astype(
                        jnp.bfloat16)
                    ctx = jnp.dot(p, vs, preferred_element_type=jnp.float32)
                    ctx_ref[pl.ds(r0, QBLK), h * DH:(h + 1) * DH] = (
                        ctx.astype(jnp.bfloat16))
            out_ref[0, pl.ds(q0, CHUNK), :] = jnp.dot(
                ctx_ref[pl.ds(q0, CHUNK), :], wo_ref[...],
                preferred_element_type=jnp.float32)

        def ring_rdma(ring, hop, src):
            dev = right if ring == 0 else left
            return pltpu.make_async_remote_copy(
                src_ref=src,
                dst_ref=comm_ref.at[ring, hop],
                send_sem=send_sems.at[ring, hop],
                recv_sem=recv_sems.at[ring, hop],
                device_id=(dev,),
                device_id_type=pl.DeviceIdType.MESH)

        def stage_and_start(s_):
            cs0 = (my - s_ + N_DEV) % N_DEV
            cs1 = (my + s_) % N_DEV
            sbuf_ref[0] = out_ref[0, pl.ds(cs0 * CHUNK, CHUNK),
                                  0:HALF].astype(jnp.bfloat16)
            sbuf_ref[1] = out_ref[0, pl.ds(cs1 * CHUNK, CHUNK),
                                  HALF:D_MODEL].astype(jnp.bfloat16)
            r0 = ring_rdma(0, s_, sbuf_ref.at[0])
            r1 = ring_rdma(1, s_, sbuf_ref.at[1])
            r0.start()
            r1.start()
            return r0, r1

        def wait_and_add(s_, r0, r1):
            r0.wait()
            r1.wait()
            cr0 = (my - s_ - 1 + N_DEV) % N_DEV
            cr1 = (my + s_ + 1) % N_DEV
            out_ref[0, pl.ds(cr0 * CHUNK, CHUNK), 0:HALF] = (
                out_ref[0, pl.ds(cr0 * CHUNK, CHUNK), 0:HALF]
                + comm_ref[0, s_].astype(jnp.float32))
            out_ref[0, pl.ds(cr1 * CHUNK, CHUNK), HALF:D_MODEL] = (
                out_ref[0, pl.ds(cr1 * CHUNK, CHUNK), HALF:D_MODEL]
                + comm_ref[1, s_].astype(jnp.float32))

        compute_chunk(my)
        h0 = stage_and_start(0)
        compute_chunk((my + 1) % N_DEV)
        compute_chunk((my + N_DEV - 1) % N_DEV)
        wait_and_add(0, *h0)
        h1 = stage_and_start(1)
        compute_chunk((my + 2) % N_DEV)
        wait_and_add(1, *h1)
        h2 = stage_and_start(2)
        wait_and_add(2, *h2)

        own0 = (my + 1) % N_DEV
        own1 = (my + N_DEV - 1) % N_DEV
        for g in range(N_DEV - 1):
            hop = (N_DEV - 1) + g
            if g == 0:
                sbuf_ref[0] = out_ref[0, pl.ds(own0 * CHUNK, CHUNK),
                                      0:HALF].astype(jnp.bfloat16)
                sbuf_ref[1] = out_ref[0, pl.ds(own1 * CHUNK, CHUNK),
                                      HALF:D_MODEL].astype(jnp.bfloat16)
                src0, src1 = sbuf_ref.at[0], sbuf_ref.at[1]
            else:
                src0 = comm_ref.at[0, hop - 1]
                src1 = comm_ref.at[1, hop - 1]
            r0 = ring_rdma(0, hop, src0)
            r1 = ring_rdma(1, hop, src1)
            r0.start()
            r1.start()
            r0.wait()
            r1.wait()
            cr0 = (my - g + N_DEV) % N_DEV
            cr1 = (my + g) % N_DEV
            out_ref[0, pl.ds(cr0 * CHUNK, CHUNK), 0:HALF] = (
                comm_ref[0, hop].astype(jnp.float32))
            out_ref[0, pl.ds(cr1 * CHUNK, CHUNK), HALF:D_MODEL] = (
                comm_ref[1, hop].astype(jnp.float32))

    out_shape = jax.ShapeDtypeStruct((1, SQ, D_MODEL), jnp.float32)
    return pl.pallas_call(
        body,
        out_shape=out_shape,
        in_specs=[pl.BlockSpec(memory_space=pltpu.VMEM)] * 5,
        out_specs=pl.BlockSpec(memory_space=pltpu.VMEM),
        scratch_shapes=[
            pltpu.VMEM((SQ, HQ * DH), jnp.bfloat16),
            pltpu.VMEM((2, N_HOPS, CHUNK, HALF), jnp.bfloat16),
            pltpu.VMEM((2, CHUNK, HALF), jnp.bfloat16),
            pltpu.SemaphoreType.DMA((2, N_HOPS)),
            pltpu.SemaphoreType.DMA((2, N_HOPS)),
        ],
        compiler_params=pltpu.CompilerParams(
            collective_id=0, vmem_limit_bytes=56 * 1024 * 1024),
    )(xb, wqb, kb, vb, wob)


# device time: 113266 ns/iter; 1.2268x vs baseline; 1.1091x over previous
import jax
import jax.numpy as jnp
from jax import lax
from jax.experimental import pallas as pl
from jax.experimental.pallas import tpu as pltpu

N_DEV = 4
HQ = 8
DH = 128
SQ = 2048
D_MODEL = 1024
WINDOW = 128
SCALE = 0.08838834764831843
CHUNK = SQ // N_DEV
KWIN = CHUNK + 2 * WINDOW
HALF = D_MODEL // 2
N_HOPS = 2 * (N_DEV - 1)


def kernel(x, Wq, K_ext, V_ext, Wo):
    xb = x[0].astype(jnp.bfloat16)
    wqb = Wq.astype(jnp.bfloat16)
    wob = Wo.astype(jnp.bfloat16)

    def body(x_ref, wq_ref, kext_ref, vext_ref, wo_ref, out_ref,
             q_ref, ctx_ref, kst_ref, vst_ref, comm_ref, sbuf_ref,
             dma_sems, send_sems, recv_sems):
        my = lax.axis_index("i")
        left = (my + N_DEV - 1) % N_DEV
        right = (my + 1) % N_DEV

        kcp = pltpu.make_async_copy(
            kext_ref.at[0, :, pl.ds(my * HQ, HQ), :], kst_ref,
            dma_sems.at[0])
        vcp = pltpu.make_async_copy(
            vext_ref.at[0, :, pl.ds(my * HQ, HQ), :], vst_ref,
            dma_sems.at[1])
        kcp.start()
        vcp.start()

        barrier = pltpu.get_barrier_semaphore()
        for nbr in (left, right):
            pl.semaphore_signal(barrier, inc=1, device_id=(nbr,),
                                device_id_type=pl.DeviceIdType.MESH)
        pl.semaphore_wait(barrier, 2)

        q_ref[...] = (jnp.dot(x_ref[...], wq_ref[...],
                              preferred_element_type=jnp.float32)
                      * SCALE).astype(jnp.bfloat16)
        kcp.wait()
        vcp.wait()

        def compute_chunk(c):
            q0 = c * CHUNK
            kw = jnp.minimum(jnp.maximum(q0 - WINDOW, 0), SQ - KWIN)
            for h in range(HQ):
                ks = kst_ref[pl.ds(kw, KWIN), h, :].astype(jnp.bfloat16)
                vs = vst_ref[pl.ds(kw, KWIN), h, :].astype(jnp.bfloat16)
                qs = q_ref[pl.ds(q0, CHUNK), h * DH:(h + 1) * DH]
                s = lax.dot_general(
                    qs, ks, (((1,), (1,)), ((), ())),
                    preferred_element_type=jnp.float32)
                ii = q0 + lax.broadcasted_iota(jnp.int32, s.shape, 0)
                jj = kw + lax.broadcasted_iota(jnp.int32, s.shape, 1)
                s = jnp.where(jnp.abs(ii - jj) <= WINDOW, s, -1e9)
                mx = jnp.max(s, axis=1, keepdims=True)
                w = jnp.exp(s - mx)
                p = (w / jnp.sum(w, axis=1, keepdims=True)).astype(
                    jnp.bfloat16)
                ctx = jnp.dot(p, vs, preferred_element_type=jnp.float32)
                ctx_ref[pl.ds(q0, CHUNK), h * DH:(h + 1) * DH] = (
                    ctx.astype(jnp.bfloat16))
            out_ref[0, pl.ds(q0, CHUNK), :] = jnp.dot(
                ctx_ref[pl.ds(q0, CHUNK), :], wo_ref[...],
                preferred_element_type=jnp.float32)

        def ring_rdma(ring, hop, src):
            dev = right if ring == 0 else left
            return pltpu.make_async_remote_copy(
                src_ref=src,
                dst_ref=comm_ref.at[ring, hop],
                send_sem=send_sems.at[ring, hop],
                recv_sem=recv_sems.at[ring, hop],
                device_id=(dev,),
                device_id_type=pl.DeviceIdType.MESH)

        def stage_and_start(s_):
            cs0 = (my - s_ + N_DEV) % N_DEV
            cs1 = (my + s_) % N_DEV
            sbuf_ref[0] = out_ref[0, pl.ds(cs0 * CHUNK, CHUNK),
                                  0:HALF].astype(jnp.bfloat16)
            sbuf_ref[1] = out_ref[0, pl.ds(cs1 * CHUNK, CHUNK),
                                  HALF:D_MODEL].astype(jnp.bfloat16)
            r0 = ring_rdma(0, s_, sbuf_ref.at[0])
            r1 = ring_rdma(1, s_, sbuf_ref.at[1])
            r0.start()
            r1.start()
            return r0, r1

        def wait_and_add(s_, r0, r1):
            r0.wait()
            r1.wait()
            cr0 = (my - s_ - 1 + N_DEV) % N_DEV
            cr1 = (my + s_ + 1) % N_DEV
            out_ref[0, pl.ds(cr0 * CHUNK, CHUNK), 0:HALF] = (
                out_ref[0, pl.ds(cr0 * CHUNK, CHUNK), 0:HALF]
                + comm_ref[0, s_].astype(jnp.float32))
            out_ref[0, pl.ds(cr1 * CHUNK, CHUNK), HALF:D_MODEL] = (
                out_ref[0, pl.ds(cr1 * CHUNK, CHUNK), HALF:D_MODEL]
                + comm_ref[1, s_].astype(jnp.float32))

        compute_chunk(my)
        h0 = stage_and_start(0)
        compute_chunk((my + 1) % N_DEV)
        compute_chunk((my + N_DEV - 1) % N_DEV)
        wait_and_add(0, *h0)
        h1 = stage_and_start(1)
        compute_chunk((my + 2) % N_DEV)
        wait_and_add(1, *h1)
        h2 = stage_and_start(2)
        wait_and_add(2, *h2)

        own0 = (my + 1) % N_DEV
        own1 = (my + N_DEV - 1) % N_DEV
        for g in range(N_DEV - 1):
            hop = (N_DEV - 1) + g
            if g == 0:
                sbuf_ref[0] = out_ref[0, pl.ds(own0 * CHUNK, CHUNK),
                                      0:HALF].astype(jnp.bfloat16)
                sbuf_ref[1] = out_ref[0, pl.ds(own1 * CHUNK, CHUNK),
                                      HALF:D_MODEL].astype(jnp.bfloat16)
                src0, src1 = sbuf_ref.at[0], sbuf_ref.at[1]
            else:
                src0 = comm_ref.at[0, hop - 1]
                src1 = comm_ref.at[1, hop - 1]
            r0 = ring_rdma(0, hop, src0)
            r1 = ring_rdma(1, hop, src1)
            r0.start()
            r1.start()
            r0.wait()
            r1.wait()
            cr0 = (my - g + N_DEV) % N_DEV
            cr1 = (my + g) % N_DEV
            out_ref[0, pl.ds(cr0 * CHUNK, CHUNK), 0:HALF] = (
                comm_ref[0, hop].astype(jnp.float32))
            out_ref[0, pl.ds(cr1 * CHUNK, CHUNK), HALF:D_MODEL] = (
                comm_ref[1, hop].astype(jnp.float32))

    out_shape = jax.ShapeDtypeStruct((1, SQ, D_MODEL), jnp.float32)
    return pl.pallas_call(
        body,
        out_shape=out_shape,
        in_specs=[
            pl.BlockSpec(memory_space=pltpu.VMEM),
            pl.BlockSpec(memory_space=pltpu.VMEM),
            pl.BlockSpec(memory_space=pl.ANY),
            pl.BlockSpec(memory_space=pl.ANY),
            pl.BlockSpec(memory_space=pltpu.VMEM),
        ],
        out_specs=pl.BlockSpec(memory_space=pltpu.VMEM),
        scratch_shapes=[
            pltpu.VMEM((SQ, HQ * DH), jnp.bfloat16),
            pltpu.VMEM((SQ, HQ * DH), jnp.bfloat16),
            pltpu.VMEM((SQ, HQ, DH), jnp.float32),
            pltpu.VMEM((SQ, HQ, DH), jnp.float32),
            pltpu.VMEM((2, N_HOPS, CHUNK, HALF), jnp.bfloat16),
            pltpu.VMEM((2, CHUNK, HALF), jnp.bfloat16),
            pltpu.SemaphoreType.DMA((2,)),
            pltpu.SemaphoreType.DMA((2, N_HOPS)),
            pltpu.SemaphoreType.DMA((2, N_HOPS)),
        ],
        compiler_params=pltpu.CompilerParams(
            collective_id=0, vmem_limit_bytes=56 * 1024 * 1024),
    )(xb, wqb, K_ext, V_ext, wob)


# device time: 106903 ns/iter; 1.2998x vs baseline; 1.0595x over previous
import jax
import jax.numpy as jnp
from jax import lax
from jax.experimental import pallas as pl
from jax.experimental.pallas import tpu as pltpu

N_DEV = 4
HQ = 8
DH = 128
SQ = 2048
D_MODEL = 1024
WINDOW = 128
SCALE = 0.08838834764831843
CHUNK = SQ // N_DEV
KWIN = CHUNK + 2 * WINDOW
HALF = D_MODEL // 2
N_HOPS = 2 * (N_DEV - 1)


def kernel(x, Wq, K_ext, V_ext, Wo):
    xb = x[0].astype(jnp.bfloat16)
    wqb = Wq.astype(jnp.bfloat16)
    wob = Wo.astype(jnp.bfloat16)

    def body(x_ref, wq_ref, kext_ref, vext_ref, wo_ref, out_ref,
             q_ref, ctx_ref, kst_ref, vst_ref, comm_ref, sbuf_ref,
             dma_sems, send_sems, recv_sems):
        my = lax.axis_index("i")
        left = (my + N_DEV - 1) % N_DEV
        right = (my + 1) % N_DEV

        kcp = pltpu.make_async_copy(
            kext_ref.at[0, :, pl.ds(my * HQ, HQ), :], kst_ref,
            dma_sems.at[0])
        vcp = pltpu.make_async_copy(
            vext_ref.at[0, :, pl.ds(my * HQ, HQ), :], vst_ref,
            dma_sems.at[1])
        kcp.start()
        vcp.start()

        barrier = pltpu.get_barrier_semaphore()
        for nbr in (left, right):
            pl.semaphore_signal(barrier, inc=1, device_id=(nbr,),
                                device_id_type=pl.DeviceIdType.MESH)
        pl.semaphore_wait(barrier, 2)

        q_ref[...] = (jnp.dot(x_ref[...], wq_ref[...],
                              preferred_element_type=jnp.float32)
                      * SCALE).astype(jnp.bfloat16)
        kcp.wait()
        vcp.wait()

        def compute_chunk(c):
            q0 = c * CHUNK
            kw = jnp.minimum(jnp.maximum(q0 - WINDOW, 0), SQ - KWIN)
            ii = q0 + lax.broadcasted_iota(jnp.int32, (CHUNK, KWIN), 0)
            jj = kw + lax.broadcasted_iota(jnp.int32, (CHUNK, KWIN), 1)
            bias = jnp.where(jnp.abs(ii - jj) <= WINDOW,
                             jnp.float32(0), jnp.float32(-1e9))
            for h in range(HQ):
                ks = kst_ref[pl.ds(kw, KWIN), h, :].astype(jnp.bfloat16)
                vs = vst_ref[pl.ds(kw, KWIN), h, :].astype(jnp.bfloat16)
                qs = q_ref[pl.ds(q0, CHUNK), h * DH:(h + 1) * DH]
                s = lax.dot_general(
                    qs, ks, (((1,), (1,)), ((), ())),
                    preferred_element_type=jnp.float32)
                w = jnp.exp(s + bias)
                wsum = jnp.sum(w, axis=1, keepdims=True)
                ctx = jnp.dot(w.astype(jnp.bfloat16), vs,
                              preferred_element_type=jnp.float32) / wsum
                ctx_ref[pl.ds(q0, CHUNK), h * DH:(h + 1) * DH] = (
                    ctx.astype(jnp.bfloat16))
            out_ref[0, pl.ds(q0, CHUNK), :] = jnp.dot(
                ctx_ref[pl.ds(q0, CHUNK), :], wo_ref[...],
                preferred_element_type=jnp.float32)

        def ring_rdma(ring, hop, src):
            dev = right if ring == 0 else left
            return pltpu.make_async_remote_copy(
                src_ref=src,
                dst_ref=comm_ref.at[ring, hop],
                send_sem=send_sems.at[ring, hop],
                recv_sem=recv_sems.at[ring, hop],
                device_id=(dev,),
                device_id_type=pl.DeviceIdType.MESH)

        def stage_and_start(s_):
            cs0 = (my - s_ + N_DEV) % N_DEV
            cs1 = (my + s_) % N_DEV
            sbuf_ref[0] = out_ref[0, pl.ds(cs0 * CHUNK, CHUNK),
                                  0:HALF].astype(jnp.bfloat16)
            sbuf_ref[1] = out_ref[0, pl.ds(cs1 * CHUNK, CHUNK),
                                  HALF:D_MODEL].astype(jnp.bfloat16)
            r0 = ring_rdma(0, s_, sbuf_ref.at[0])
            r1 = ring_rdma(1, s_, sbuf_ref.at[1])
            r0.start()
            r1.start()
            return r0, r1

        def wait_and_add(s_, r0, r1):
            r0.wait()
            r1.wait()
            cr0 = (my - s_ - 1 + N_DEV) % N_DEV
            cr1 = (my + s_ + 1) % N_DEV
            out_ref[0, pl.ds(cr0 * CHUNK, CHUNK), 0:HALF] = (
                out_ref[0, pl.ds(cr0 * CHUNK, CHUNK), 0:HALF]
                + comm_ref[0, s_].astype(jnp.float32))
            out_ref[0, pl.ds(cr1 * CHUNK, CHUNK), HALF:D_MODEL] = (
                out_ref[0, pl.ds(cr1 * CHUNK, CHUNK), HALF:D_MODEL]
                + comm_ref[1, s_].astype(jnp.float32))

        compute_chunk(my)
        h0 = stage_and_start(0)
        compute_chunk((my + 1) % N_DEV)
        compute_chunk((my + N_DEV - 1) % N_DEV)
        wait_and_add(0, *h0)
        h1 = stage_and_start(1)
        compute_chunk((my + 2) % N_DEV)
        wait_and_add(1, *h1)
        h2 = stage_and_start(2)
        wait_and_add(2, *h2)

        own0 = (my + 1) % N_DEV
        own1 = (my + N_DEV - 1) % N_DEV
        for g in range(N_DEV - 1):
            hop = (N_DEV - 1) + g
            if g == 0:
                sbuf_ref[0] = out_ref[0, pl.ds(own0 * CHUNK, CHUNK),
                                      0:HALF].astype(jnp.bfloat16)
                sbuf_ref[1] = out_ref[0, pl.ds(own1 * CHUNK, CHUNK),
                                      HALF:D_MODEL].astype(jnp.bfloat16)
                src0, src1 = sbuf_ref.at[0], sbuf_ref.at[1]
            else:
                src0 = comm_ref.at[0, hop - 1]
                src1 = comm_ref.at[1, hop - 1]
            r0 = ring_rdma(0, hop, src0)
            r1 = ring_rdma(1, hop, src1)
            r0.start()
            r1.start()
            r0.wait()
            r1.wait()
            cr0 = (my - g + N_DEV) % N_DEV
            cr1 = (my + g) % N_DEV
            out_ref[0, pl.ds(cr0 * CHUNK, CHUNK), 0:HALF] = (
                comm_ref[0, hop].astype(jnp.float32))
            out_ref[0, pl.ds(cr1 * CHUNK, CHUNK), HALF:D_MODEL] = (
                comm_ref[1, hop].astype(jnp.float32))

    out_shape = jax.ShapeDtypeStruct((1, SQ, D_MODEL), jnp.float32)
    return pl.pallas_call(
        body,
        out_shape=out_shape,
        in_specs=[
            pl.BlockSpec(memory_space=pltpu.VMEM),
            pl.BlockSpec(memory_space=pltpu.VMEM),
            pl.BlockSpec(memory_space=pl.ANY),
            pl.BlockSpec(memory_space=pl.ANY),
            pl.BlockSpec(memory_space=pltpu.VMEM),
        ],
        out_specs=pl.BlockSpec(memory_space=pltpu.VMEM),
        scratch_shapes=[
            pltpu.VMEM((SQ, HQ * DH), jnp.bfloat16),
            pltpu.VMEM((SQ, HQ * DH), jnp.bfloat16),
            pltpu.VMEM((SQ, HQ, DH), jnp.float32),
            pltpu.VMEM((SQ, HQ, DH), jnp.float32),
            pltpu.VMEM((2, N_HOPS, CHUNK, HALF), jnp.bfloat16),
            pltpu.VMEM((2, CHUNK, HALF), jnp.bfloat16),
            pltpu.SemaphoreType.DMA((2,)),
            pltpu.SemaphoreType.DMA((2, N_HOPS)),
            pltpu.SemaphoreType.DMA((2, N_HOPS)),
        ],
        compiler_params=pltpu.CompilerParams(
            collective_id=0, vmem_limit_bytes=56 * 1024 * 1024),
    )(xb, wqb, K_ext, V_ext, wob)
